# baseline (speedup 1.0000x reference)
import functools

import numpy as np
import jax
import jax.numpy as jnp
from jax import lax
from jax.experimental import pallas as pl
from jax.experimental.pallas import tpu as pltpu
from jax.experimental.pallas import tpu_sc as plsc
from jax.experimental.layout import Layout, with_layout_constraint

_F = 26
_ROWS = 4000
_TOT = _F * _ROWS
_D = 16
_B = 4096
_NP = _F * (_F - 1) // 2
_NPP = 336
_NBLK = _NPP // 16
_BN_S = float(1.0 / np.sqrt(1.0 + 1e-5))

_OFF = np.arange(_F, dtype=np.int32) * _ROWS
_I, _J = np.triu_indices(_F, k=1)
_IP = np.concatenate([_I, np.zeros(_NPP - _NP, np.int64)]).astype(np.int32)
_JP = np.concatenate([_J, np.ones(_NPP - _NP, np.int64)]).astype(np.int32)
_PA = ((_JP - 1) * _F + _IP).astype(np.int32)
_PB = (_IP * _F + _JP).astype(np.int32)
_PAIRS_FLAT = np.concatenate([_PA, _PB])

_NB = 4
_CH = _NB * _F * _F
_WB = _B // 32
_NCH = _WB // _NB


def _sc_gather_inter(tflat, idx_full, pairs_flat):
    info = plsc.get_sparse_core_info()
    nc, ns = info.num_cores, info.num_subcores
    mesh = plsc.VectorSubcoreMesh(core_axis_name="c", subcore_axis_name="s")
    out_type = (
        jax.ShapeDtypeStruct((_B * _F, _D), jnp.float32),
        jax.ShapeDtypeStruct((_B * _NPP,), jnp.float32),
    )

    @functools.partial(
        pl.kernel,
        mesh=mesh,
        out_type=out_type,
        compiler_params=pltpu.CompilerParams(use_tc_tiling_on_sc=False,
                                             needs_layout_passes=False),
        scratch_types=[
            pltpu.VMEM((_CH,), jnp.int32),
            pltpu.VMEM((_CH,), jnp.int32),
            pltpu.VMEM((_CH, _D), jnp.float32),
            pltpu.VMEM((_CH, _D), jnp.float32),
            pltpu.VMEM((_NB * _NPP,), jnp.float32),
            pltpu.VMEM((_NB * _NPP,), jnp.float32),
            pltpu.VMEM((2 * _NPP,), jnp.int32),
            pltpu.SemaphoreType.DMA,
            pltpu.SemaphoreType.DMA,
        ],
    )
    def k(tf_hbm, if_hbm, pf_hbm, ro_hbm, io_hbm, idx_a, idx_b, rows_a,
          rows_b, int_a, int_b, pf_v, sem_a, sem_b):
        wid = lax.axis_index("s") * nc + lax.axis_index("c")
        pltpu.sync_copy(pf_hbm, pf_v)
        row_base = wid * _WB * _F * _F

        def issue(c, idx_v, rows_v, sem):
            off = row_base + c * _CH
            pltpu.sync_copy(if_hbm.at[pl.ds(off, _CH)], idx_v)
            return pltpu.async_copy(tf_hbm.at[idx_v], rows_v, sem)

        def process(c, rows_v, int_v):
            def bat(bi, carry):
                roff = bi * _F * _F
                for kb in range(_NBLK):
                    ra = pf_v[pl.ds(kb * 16, 16)] + roff
                    rb = pf_v[pl.ds(_NPP + kb * 16, 16)] + roff
                    acc = jnp.zeros((16,), jnp.float32)
                    for d in range(_D):
                        cols = jnp.full((16,), d, jnp.int32)
                        va = plsc.load_gather(rows_v, [ra, cols])
                        vb = plsc.load_gather(rows_v, [rb, cols])
                        acc = acc + va * vb
                    int_v[pl.ds(bi * _NPP + kb * 16, 16)] = acc
                return carry

            lax.fori_loop(0, _NB, bat, 0)
            b0 = wid * _WB + c * _NB
            for bi in range(_NB):
                pltpu.sync_copy(
                    rows_v.at[pl.ds(bi * _F * _F + (_F - 1) * _F, _F)],
                    ro_hbm.at[pl.ds((b0 + bi) * _F, _F)])
            pltpu.sync_copy(int_v, io_hbm.at[pl.ds(b0 * _NPP, _NB * _NPP)])

        def body(kk, carry):
            issue(kk, idx_a, rows_a, sem_a).wait()
            process(kk, rows_a, int_a)
            return carry

        lax.fori_loop(0, _NCH, body, 0)

    return k(tflat, idx_full, pairs_flat)


def _tc_body(raw_ref, int_ref, w0t_ref, w0b_ref, b0_ref, w1_ref, b1_ref,
             w2_ref, b2_ref, w3_ref, b3_ref, out_ref):
    h = jnp.dot(raw_ref[...], w0t_ref[...], preferred_element_type=jnp.float32)
    h = h + jnp.dot(int_ref[...], w0b_ref[...],
                    preferred_element_type=jnp.float32)
    h = jnp.maximum(h + b0_ref[...], 0.0)
    h = jnp.dot(h, w1_ref[...], preferred_element_type=jnp.float32)
    h = jnp.maximum(h + b1_ref[...], 0.0)
    h = jnp.dot(h, w2_ref[...], preferred_element_type=jnp.float32)
    h = jnp.maximum(h + b2_ref[...], 0.0)
    o = jnp.dot(h, w3_ref[...], preferred_element_type=jnp.float32)
    out_ref[...] = jax.nn.sigmoid(o + b3_ref[...])


def _tc_mlp(raw, inter, w0t, w0b, b0, w1, b1, w2, b2, w3, b3):
    blk = 256
    grid = (_B // blk,)

    def full(arr):
        return pl.BlockSpec(arr.shape, lambda i: (0,) * arr.ndim)

    return pl.pallas_call(
        _tc_body,
        grid=grid,
        in_specs=[
            pl.BlockSpec((blk, _F * _D), lambda i: (i, 0)),
            pl.BlockSpec((blk, _NPP), lambda i: (i, 0)),
            full(w0t), full(w0b), full(b0), full(w1), full(b1),
            full(w2), full(b2), full(w3), full(b3),
        ],
        out_specs=pl.BlockSpec((blk, 1), lambda i: (i, 0)),
        out_shape=jax.ShapeDtypeStruct((_B, 1), jnp.float32),
    )(raw, inter, w0t, w0b, b0, w1, b1, w2, b2, w3, b3)


def kernel(x, tables, W0, b0, g0, bb0, W1, b1, g1, bb1, W2, b2, g2, bb2,
           W3, b3):
    tmp = x + jnp.asarray(_OFF)[None, :]
    idx_full = (tmp[:, None, :]
                + (jnp.arange(_F, dtype=jnp.int32) * _TOT)[None, :, None]
                ).reshape(-1)

    tflat = with_layout_constraint(
        tables.reshape(_F * _TOT, _D),
        Layout(major_to_minor=(0, 1), tiling=((16,),)))
    raw_r, int_r = _sc_gather_inter(tflat, idx_full,
                                    jnp.asarray(_PAIRS_FLAT))
    raw2 = raw_r.reshape(_B, _F * _D)
    int2 = int_r.reshape(_B, _NPP)

    def fold(w, bias, g, bb):
        s = g * _BN_S
        return w * s[None, :], bias * s + bb

    w0f, b0f = fold(W0, b0, g0, bb0)
    w1f, b1f = fold(W1, b1, g1, bb1)
    w2f, b2f = fold(W2, b2, g2, bb2)
    w0t = w0f[: _F * _D]
    w0b = jnp.concatenate(
        [w0f[_F * _D:], jnp.zeros((_NPP - _NP, w0f.shape[1]), jnp.float32)])

    out = _tc_mlp(raw2, int2, w0t, w0b, b0f[None, :], w1f, b1f[None, :],
                  w2f, b2f[None, :], W3, b3[None, :])
    return out.reshape(_B)


# speedup vs baseline: 1.5643x; 1.5643x over previous
import functools

import numpy as np
import jax
import jax.numpy as jnp
from jax import lax
from jax.experimental import pallas as pl
from jax.experimental.pallas import tpu as pltpu
from jax.experimental.pallas import tpu_sc as plsc
from jax.experimental.layout import Layout, with_layout_constraint

_F = 26
_ROWS = 4000
_TOT = _F * _ROWS
_D = 16
_B = 4096
_NP = _F * (_F - 1) // 2
_NPP = 336
_NBLK = _NPP // 16
_BN_S = float(1.0 / np.sqrt(1.0 + 1e-5))

_OFF = np.arange(_F, dtype=np.int32) * _ROWS
_I, _J = np.triu_indices(_F, k=1)
_IP = np.concatenate([_I, np.zeros(_NPP - _NP, np.int64)]).astype(np.int32)
_JP = np.concatenate([_J, np.ones(_NPP - _NP, np.int64)]).astype(np.int32)
_PAIRS_FLAT = np.concatenate([
    (_JP - 1), _IP, _IP, _JP]).astype(np.int32)

_NB = 4
_WB = _B // 32


def _sc_gather_inter(table, tmp_idx, pairs_flat):
    info = plsc.get_sparse_core_info()
    nc, ns = info.num_cores, info.num_subcores
    nw = nc * ns
    mesh = plsc.VectorSubcoreMesh(core_axis_name="c", subcore_axis_name="s")
    out_type = (
        jax.ShapeDtypeStruct((_B * _F, _D), jnp.float32),
        jax.ShapeDtypeStruct((_B * _NPP,), jnp.float32),
    )

    @functools.partial(
        pl.kernel,
        mesh=mesh,
        out_type=out_type,
        compiler_params=pltpu.CompilerParams(use_tc_tiling_on_sc=False,
                                             needs_layout_passes=False),
        scratch_types=[
            pltpu.VMEM((_NB * _F,), jnp.int32),
            pltpu.VMEM((_F, _NB * _F, _D), jnp.float32),
            pltpu.VMEM((_NB * _NPP,), jnp.float32),
            pltpu.VMEM((4 * _NPP,), jnp.int32),
            pltpu.SemaphoreType.DMA,
        ],
    )
    def k(table_hbm, ti_hbm, pf_hbm, ro_hbm, io_hbm, idx_v, rows_v, int_v,
          pf_v, sem):
        wid = lax.axis_index("s") * nc + lax.axis_index("c")
        bbase = wid * _WB
        pltpu.sync_copy(pf_hbm, pf_v)

        def chunk(c, carry):
            b0 = bbase + c * _NB
            pltpu.sync_copy(ti_hbm.at[pl.ds(b0 * _F, _NB * _F)], idx_v)
            copies = [
                pltpu.async_copy(table_hbm.at[t].at[idx_v], rows_v.at[t], sem)
                for t in range(_F)
            ]
            for cp in copies:
                cp.wait()

            def bat(bi, carry2):
                roff = bi * _F
                for kb in range(_NBLK):
                    ta = pf_v[pl.ds(kb * 16, 16)]
                    ra = pf_v[pl.ds(_NPP + kb * 16, 16)] + roff
                    tb = pf_v[pl.ds(2 * _NPP + kb * 16, 16)]
                    rb = pf_v[pl.ds(3 * _NPP + kb * 16, 16)] + roff
                    acc = jnp.zeros((16,), jnp.float32)
                    for d in range(_D):
                        cols = jnp.full((16,), d, jnp.int32)
                        va = plsc.load_gather(rows_v, [ta, ra, cols])
                        vb = plsc.load_gather(rows_v, [tb, rb, cols])
                        acc = acc + va * vb
                    int_v[pl.ds(bi * _NPP + kb * 16, 16)] = acc
                return carry2

            lax.fori_loop(0, _NB, bat, 0)
            pltpu.sync_copy(rows_v.at[_F - 1],
                            ro_hbm.at[pl.ds(b0 * _F, _NB * _F)])
            pltpu.sync_copy(int_v, io_hbm.at[pl.ds(b0 * _NPP, _NB * _NPP)])
            return carry

        lax.fori_loop(0, _WB // _NB, chunk, 0)

    return k(table, tmp_idx, pairs_flat)


def _tc_body(raw_ref, int_ref, w0t_ref, w0b_ref, b0_ref, w1_ref, b1_ref,
             w2_ref, b2_ref, w3_ref, b3_ref, out_ref):
    h = jnp.dot(raw_ref[...], w0t_ref[...], preferred_element_type=jnp.float32)
    h = h + jnp.dot(int_ref[...], w0b_ref[...],
                    preferred_element_type=jnp.float32)
    h = jnp.maximum(h + b0_ref[...], 0.0)
    h = jnp.dot(h, w1_ref[...], preferred_element_type=jnp.float32)
    h = jnp.maximum(h + b1_ref[...], 0.0)
    h = jnp.dot(h, w2_ref[...], preferred_element_type=jnp.float32)
    h = jnp.maximum(h + b2_ref[...], 0.0)
    o = jnp.dot(h, w3_ref[...], preferred_element_type=jnp.float32)
    out_ref[...] = jax.nn.sigmoid(o + b3_ref[...])


def _tc_mlp(raw, inter, w0t, w0b, b0, w1, b1, w2, b2, w3, b3):
    blk = 256
    grid = (_B // blk,)

    def full(arr):
        return pl.BlockSpec(arr.shape, lambda i: (0,) * arr.ndim)

    return pl.pallas_call(
        _tc_body,
        grid=grid,
        in_specs=[
            pl.BlockSpec((blk, _F * _D), lambda i: (i, 0)),
            pl.BlockSpec((blk, _NPP), lambda i: (i, 0)),
            full(w0t), full(w0b), full(b0), full(w1), full(b1),
            full(w2), full(b2), full(w3), full(b3),
        ],
        out_specs=pl.BlockSpec((blk, 1), lambda i: (i, 0)),
        out_shape=jax.ShapeDtypeStruct((_B, 1), jnp.float32),
    )(raw, inter, w0t, w0b, b0, w1, b1, w2, b2, w3, b3)


def kernel(x, tables, W0, b0, g0, bb0, W1, b1, g1, bb1, W2, b2, g2, bb2,
           W3, b3):
    tmp_idx = (x + jnp.asarray(_OFF)[None, :]).reshape(-1)

    tables_c = with_layout_constraint(
        tables, Layout(major_to_minor=(0, 1, 2), tiling=((16,),)))
    raw_r, int_r = _sc_gather_inter(tables_c, tmp_idx,
                                    jnp.asarray(_PAIRS_FLAT))
    raw2 = raw_r.reshape(_B, _F * _D)
    int2 = int_r.reshape(_B, _NPP)

    def fold(w, bias, g, bb):
        s = g * _BN_S
        return w * s[None, :], bias * s + bb

    w0f, b0f = fold(W0, b0, g0, bb0)
    w1f, b1f = fold(W1, b1, g1, bb1)
    w2f, b2f = fold(W2, b2, g2, bb2)
    w0t = w0f[: _F * _D]
    w0b = jnp.concatenate(
        [w0f[_F * _D:], jnp.zeros((_NPP - _NP, w0f.shape[1]), jnp.float32)])

    out = _tc_mlp(raw2, int2, w0t, w0b, b0f[None, :], w1f, b1f[None, :],
                  w2f, b2f[None, :], W3, b3[None, :])
    return out.reshape(_B)


# speedup vs baseline: 1.5889x; 1.0157x over previous
import functools

import numpy as np
import jax
import jax.numpy as jnp
from jax import lax
from jax.experimental import pallas as pl
from jax.experimental.pallas import tpu as pltpu
from jax.experimental.pallas import tpu_sc as plsc
from jax.experimental.layout import Layout, with_layout_constraint

_F = 26
_ROWS = 4000
_TOT = _F * _ROWS
_D = 16
_B = 4096
_NP = _F * (_F - 1) // 2
_NPP = 336
_NBLK = _NPP // 16
_BN_S = float(1.0 / np.sqrt(1.0 + 1e-5))

_OFF = np.arange(_F, dtype=np.int32) * _ROWS
_I, _J = np.triu_indices(_F, k=1)
_IP = np.concatenate([_I, np.zeros(_NPP - _NP, np.int64)]).astype(np.int32)
_JP = np.concatenate([_J, np.ones(_NPP - _NP, np.int64)]).astype(np.int32)
_PAIRS_FLAT = np.concatenate([
    (_JP - 1), _IP, _IP, _JP]).astype(np.int32)

_NB = 8
_WB = _B // 32


def _sc_gather_inter(table, tmp_idx, pairs_flat):
    info = plsc.get_sparse_core_info()
    nc, ns = info.num_cores, info.num_subcores
    nw = nc * ns
    mesh = plsc.VectorSubcoreMesh(core_axis_name="c", subcore_axis_name="s")
    out_type = (
        jax.ShapeDtypeStruct((_B * _F, _D), jnp.float32),
        jax.ShapeDtypeStruct((_B * _NPP,), jnp.float32),
    )

    @functools.partial(
        pl.kernel,
        mesh=mesh,
        out_type=out_type,
        compiler_params=pltpu.CompilerParams(use_tc_tiling_on_sc=False,
                                             needs_layout_passes=False),
        scratch_types=[
            pltpu.VMEM((_NB * _F,), jnp.int32),
            pltpu.VMEM((_F, _NB * _F, _D), jnp.float32),
            pltpu.VMEM((_NB * _NPP,), jnp.float32),
            pltpu.VMEM((4 * _NPP,), jnp.int32),
            pltpu.SemaphoreType.DMA,
        ],
    )
    def k(table_hbm, ti_hbm, pf_hbm, ro_hbm, io_hbm, idx_v, rows_v, int_v,
          pf_v, sem):
        wid = lax.axis_index("s") * nc + lax.axis_index("c")
        bbase = wid * _WB
        pltpu.sync_copy(pf_hbm, pf_v)

        def chunk(c, carry):
            b0 = bbase + c * _NB
            pltpu.sync_copy(ti_hbm.at[pl.ds(b0 * _F, _NB * _F)], idx_v)
            copies = [
                pltpu.async_copy(table_hbm.at[t].at[idx_v], rows_v.at[t], sem)
                for t in range(_F)
            ]
            for cp in copies:
                cp.wait()

            def bat(bi, carry2):
                roff = bi * _F
                for kb in range(_NBLK):
                    ta = pf_v[pl.ds(kb * 16, 16)]
                    ra = pf_v[pl.ds(_NPP + kb * 16, 16)] + roff
                    tb = pf_v[pl.ds(2 * _NPP + kb * 16, 16)]
                    rb = pf_v[pl.ds(3 * _NPP + kb * 16, 16)] + roff
                    acc = jnp.zeros((16,), jnp.float32)
                    for d in range(_D):
                        cols = jnp.full((16,), d, jnp.int32)
                        va = plsc.load_gather(rows_v, [ta, ra, cols])
                        vb = plsc.load_gather(rows_v, [tb, rb, cols])
                        acc = acc + va * vb
                    int_v[pl.ds(bi * _NPP + kb * 16, 16)] = acc
                return carry2

            lax.fori_loop(0, _NB, bat, 0)
            pltpu.sync_copy(rows_v.at[_F - 1],
                            ro_hbm.at[pl.ds(b0 * _F, _NB * _F)])
            pltpu.sync_copy(int_v, io_hbm.at[pl.ds(b0 * _NPP, _NB * _NPP)])
            return carry

        lax.fori_loop(0, _WB // _NB, chunk, 0)

    return k(table, tmp_idx, pairs_flat)


def _tc_body(raw_ref, int_ref, w0t_ref, w0b_ref, b0_ref, w1_ref, b1_ref,
             w2_ref, b2_ref, w3_ref, b3_ref, out_ref):
    h = jnp.dot(raw_ref[...], w0t_ref[...], preferred_element_type=jnp.float32)
    h = h + jnp.dot(int_ref[...], w0b_ref[...],
                    preferred_element_type=jnp.float32)
    h = jnp.maximum(h + b0_ref[...], 0.0)
    h = jnp.dot(h, w1_ref[...], preferred_element_type=jnp.float32)
    h = jnp.maximum(h + b1_ref[...], 0.0)
    h = jnp.dot(h, w2_ref[...], preferred_element_type=jnp.float32)
    h = jnp.maximum(h + b2_ref[...], 0.0)
    o = jnp.dot(h, w3_ref[...], preferred_element_type=jnp.float32)
    out_ref[...] = jax.nn.sigmoid(o + b3_ref[...])


def _tc_mlp(raw, inter, w0t, w0b, b0, w1, b1, w2, b2, w3, b3):
    blk = 256
    grid = (_B // blk,)

    def full(arr):
        return pl.BlockSpec(arr.shape, lambda i: (0,) * arr.ndim)

    return pl.pallas_call(
        _tc_body,
        grid=grid,
        in_specs=[
            pl.BlockSpec((blk, _F * _D), lambda i: (i, 0)),
            pl.BlockSpec((blk, _NPP), lambda i: (i, 0)),
            full(w0t), full(w0b), full(b0), full(w1), full(b1),
            full(w2), full(b2), full(w3), full(b3),
        ],
        out_specs=pl.BlockSpec((blk, 1), lambda i: (i, 0)),
        out_shape=jax.ShapeDtypeStruct((_B, 1), jnp.float32),
    )(raw, inter, w0t, w0b, b0, w1, b1, w2, b2, w3, b3)


def kernel(x, tables, W0, b0, g0, bb0, W1, b1, g1, bb1, W2, b2, g2, bb2,
           W3, b3):
    tmp_idx = (x + jnp.asarray(_OFF)[None, :]).reshape(-1)

    tables_c = with_layout_constraint(
        tables, Layout(major_to_minor=(0, 1, 2), tiling=((16,),)))
    raw_r, int_r = _sc_gather_inter(tables_c, tmp_idx,
                                    jnp.asarray(_PAIRS_FLAT))
    raw2 = raw_r.reshape(_B, _F * _D)
    int2 = int_r.reshape(_B, _NPP)

    def fold(w, bias, g, bb):
        s = g * _BN_S
        return w * s[None, :], bias * s + bb

    w0f, b0f = fold(W0, b0, g0, bb0)
    w1f, b1f = fold(W1, b1, g1, bb1)
    w2f, b2f = fold(W2, b2, g2, bb2)
    w0t = w0f[: _F * _D]
    w0b = jnp.concatenate(
        [w0f[_F * _D:], jnp.zeros((_NPP - _NP, w0f.shape[1]), jnp.float32)])

    out = _tc_mlp(raw2, int2, w0t, w0b, b0f[None, :], w1f, b1f[None, :],
                  w2f, b2f[None, :], W3, b3[None, :])
    return out.reshape(_B)


# speedup vs baseline: 1.5961x; 1.0045x over previous
import functools

import numpy as np
import jax
import jax.numpy as jnp
from jax import lax
from jax.experimental import pallas as pl
from jax.experimental.pallas import tpu as pltpu
from jax.experimental.pallas import tpu_sc as plsc
from jax.experimental.layout import Layout, with_layout_constraint

_F = 26
_ROWS = 4000
_TOT = _F * _ROWS
_D = 16
_B = 4096
_NP = _F * (_F - 1) // 2
_NPP = 336
_NBLK = _NPP // 16
_BN_S = float(1.0 / np.sqrt(1.0 + 1e-5))

_OFF = np.arange(_F, dtype=np.int32) * _ROWS
_I, _J = np.triu_indices(_F, k=1)
_IP = np.concatenate([_I, np.zeros(_NPP - _NP, np.int64)]).astype(np.int32)
_JP = np.concatenate([_J, np.ones(_NPP - _NP, np.int64)]).astype(np.int32)
_NB = 8
_PA16 = (((_JP - 1) * (_NB * _F) + _IP) * _D).astype(np.int32)
_PB16 = ((_IP * (_NB * _F) + _JP) * _D).astype(np.int32)
_PAIRS_FLAT = np.concatenate([_PA16, _PB16])
_WB = _B // 32


def _sc_gather_inter(table, tmp_idx, pairs_flat):
    info = plsc.get_sparse_core_info()
    nc, ns = info.num_cores, info.num_subcores
    nw = nc * ns
    mesh = plsc.VectorSubcoreMesh(core_axis_name="c", subcore_axis_name="s")
    out_type = (
        jax.ShapeDtypeStruct((_B * _F, _D), jnp.float32),
        jax.ShapeDtypeStruct((_B * _NPP,), jnp.float32),
    )

    @functools.partial(
        pl.kernel,
        mesh=mesh,
        out_type=out_type,
        compiler_params=pltpu.CompilerParams(use_tc_tiling_on_sc=False,
                                             needs_layout_passes=False),
        scratch_types=[
            pltpu.VMEM((_NB * _F,), jnp.int32),
            pltpu.VMEM((_F, _NB * _F, _D), jnp.float32),
            pltpu.VMEM((_NB * _NPP,), jnp.float32),
            pltpu.VMEM((2 * _NPP,), jnp.int32),
            pltpu.SemaphoreType.DMA,
        ],
    )
    def k(table_hbm, ti_hbm, pf_hbm, ro_hbm, io_hbm, idx_v, rows_v, int_v,
          pf_v, sem):
        wid = lax.axis_index("s") * nc + lax.axis_index("c")
        bbase = wid * _WB
        pltpu.sync_copy(pf_hbm, pf_v)

        def chunk(c, carry):
            b0 = bbase + c * _NB
            pltpu.sync_copy(ti_hbm.at[pl.ds(b0 * _F, _NB * _F)], idx_v)
            copies = [
                pltpu.async_copy(table_hbm.at[t].at[idx_v], rows_v.at[t], sem)
                for t in range(_F)
            ]
            for cp in copies:
                cp.wait()

            def bat(bi, carry2):
                boff = bi * (_F * _D)
                zz = jnp.zeros((16,), jnp.int32)
                for kb in range(_NBLK):
                    ca = pf_v[pl.ds(kb * 16, 16)] + boff
                    cb = pf_v[pl.ds(_NPP + kb * 16, 16)] + boff
                    acc = jnp.zeros((16,), jnp.float32)
                    for d in range(_D):
                        va = plsc.load_gather(rows_v, [zz, zz, ca + d])
                        vb = plsc.load_gather(rows_v, [zz, zz, cb + d])
                        acc = acc + va * vb
                    int_v[pl.ds(bi * _NPP + kb * 16, 16)] = acc
                return carry2

            lax.fori_loop(0, _NB, bat, 0)
            pltpu.sync_copy(rows_v.at[_F - 1],
                            ro_hbm.at[pl.ds(b0 * _F, _NB * _F)])
            pltpu.sync_copy(int_v, io_hbm.at[pl.ds(b0 * _NPP, _NB * _NPP)])
            return carry

        lax.fori_loop(0, _WB // _NB, chunk, 0)

    return k(table, tmp_idx, pairs_flat)


def _tc_body(raw_ref, int_ref, w0t_ref, w0b_ref, b0_ref, w1_ref, b1_ref,
             w2_ref, b2_ref, w3_ref, b3_ref, out_ref):
    h = jnp.dot(raw_ref[...], w0t_ref[...], preferred_element_type=jnp.float32)
    h = h + jnp.dot(int_ref[...], w0b_ref[...],
                    preferred_element_type=jnp.float32)
    h = jnp.maximum(h + b0_ref[...], 0.0)
    h = jnp.dot(h, w1_ref[...], preferred_element_type=jnp.float32)
    h = jnp.maximum(h + b1_ref[...], 0.0)
    h = jnp.dot(h, w2_ref[...], preferred_element_type=jnp.float32)
    h = jnp.maximum(h + b2_ref[...], 0.0)
    o = jnp.dot(h, w3_ref[...], preferred_element_type=jnp.float32)
    out_ref[...] = jax.nn.sigmoid(o + b3_ref[...])


def _tc_mlp(raw, inter, w0t, w0b, b0, w1, b1, w2, b2, w3, b3):
    blk = 256
    grid = (_B // blk,)

    def full(arr):
        return pl.BlockSpec(arr.shape, lambda i: (0,) * arr.ndim)

    return pl.pallas_call(
        _tc_body,
        grid=grid,
        in_specs=[
            pl.BlockSpec((blk, _F * _D), lambda i: (i, 0)),
            pl.BlockSpec((blk, _NPP), lambda i: (i, 0)),
            full(w0t), full(w0b), full(b0), full(w1), full(b1),
            full(w2), full(b2), full(w3), full(b3),
        ],
        out_specs=pl.BlockSpec((blk, 1), lambda i: (i, 0)),
        out_shape=jax.ShapeDtypeStruct((_B, 1), jnp.float32),
    )(raw, inter, w0t, w0b, b0, w1, b1, w2, b2, w3, b3)


def kernel(x, tables, W0, b0, g0, bb0, W1, b1, g1, bb1, W2, b2, g2, bb2,
           W3, b3):
    tmp_idx = (x + jnp.asarray(_OFF)[None, :]).reshape(-1)

    tables_c = with_layout_constraint(
        tables, Layout(major_to_minor=(0, 1, 2), tiling=((16,),)))
    raw_r, int_r = _sc_gather_inter(tables_c, tmp_idx,
                                    jnp.asarray(_PAIRS_FLAT))
    raw2 = raw_r.reshape(_B, _F * _D)
    int2 = int_r.reshape(_B, _NPP)

    def fold(w, bias, g, bb):
        s = g * _BN_S
        return w * s[None, :], bias * s + bb

    w0f, b0f = fold(W0, b0, g0, bb0)
    w1f, b1f = fold(W1, b1, g1, bb1)
    w2f, b2f = fold(W2, b2, g2, bb2)
    w0t = w0f[: _F * _D]
    w0b = jnp.concatenate(
        [w0f[_F * _D:], jnp.zeros((_NPP - _NP, w0f.shape[1]), jnp.float32)])

    out = _tc_mlp(raw2, int2, w0t, w0b, b0f[None, :], w1f, b1f[None, :],
                  w2f, b2f[None, :], W3, b3[None, :])
    return out.reshape(_B)
